# baseline (speedup 1.0000x reference)
import functools

import jax
import jax.numpy as jnp
from jax import lax
from jax.experimental import pallas as pl
from jax.experimental.pallas import tpu as pltpu
from jax.experimental.pallas import tpu_sc as plsc

_NC = 2
_NS = 16
_NW = _NC * _NS


def _broadcast_table(w, b):
    t, d = w.shape
    rows = t // _NW

    mesh = plsc.VectorSubcoreMesh(core_axis_name="c", subcore_axis_name="s")

    @functools.partial(
        pl.kernel,
        mesh=mesh,
        out_type=jax.ShapeDtypeStruct((b, t, d), jnp.float32),
        scratch_types=[pltpu.SemaphoreType.DMA] * b,
    )
    def k(w_hbm, out_hbm, *sems):
        wid = lax.axis_index("s") * _NC + lax.axis_index("c")
        base = wid * rows
        src = w_hbm.at[pl.ds(base, rows)]
        copies = [
            pltpu.async_copy(src, out_hbm.at[i, pl.ds(base, rows)], sems[i])
            for i in range(b)
        ]
        for c in copies:
            c.wait()

    return k(w)


def kernel(x, embed_weight):
    b = x.shape[0]
    return _broadcast_table(embed_weight, b)


# speedup vs baseline: 25.1571x; 25.1571x over previous
import functools

import jax
import jax.numpy as jnp
from jax import lax
from jax.experimental import pallas as pl
from jax.experimental.pallas import tpu as pltpu
from jax.experimental.pallas import tpu_sc as plsc

_NC = 2


def _broadcast_table(w, b):
    t, d = w.shape
    half = t // _NC

    mesh = plsc.ScalarSubcoreMesh(axis_name="c", num_cores=_NC)

    @functools.partial(
        pl.kernel,
        mesh=mesh,
        out_type=jax.ShapeDtypeStruct((b, t, d), jnp.float32),
        scratch_types=[
            pltpu.VMEM_SHARED((half, d), jnp.float32),
        ] + [pltpu.SemaphoreType.DMA] * b,
    )
    def k(w_hbm, out_hbm, stage, *sems):
        c = lax.axis_index("c")
        base = c * half
        pltpu.sync_copy(w_hbm.at[pl.ds(base, half)], stage)
        copies = [
            pltpu.async_copy(stage, out_hbm.at[i, pl.ds(base, half)], sems[i])
            for i in range(b)
        ]
        for cp in copies:
            cp.wait()

    return k(w)


def kernel(x, embed_weight):
    b = x.shape[0]
    return _broadcast_table(embed_weight, b)


# speedup vs baseline: 32.1572x; 1.2783x over previous
import functools

import jax
import jax.numpy as jnp
from jax import lax
from jax.experimental import pallas as pl
from jax.experimental.pallas import tpu as pltpu
from jax.experimental.pallas import tpu_sc as plsc

_NC = 2
_NS = 16
_NW = _NC * _NS


def _broadcast_table(w, b):
    t, d = w.shape
    rows = t // _NW

    mesh = plsc.VectorSubcoreMesh(core_axis_name="c", subcore_axis_name="s")

    @functools.partial(
        pl.kernel,
        mesh=mesh,
        out_type=jax.ShapeDtypeStruct((b, t, d), jnp.float32),
        scratch_types=[
            pltpu.VMEM((rows, d), jnp.float32),
        ] + [pltpu.SemaphoreType.DMA] * b,
    )
    def k(w_hbm, out_hbm, buf, *sems):
        wid = lax.axis_index("s") * _NC + lax.axis_index("c")
        base = wid * rows
        pltpu.sync_copy(w_hbm.at[pl.ds(base, rows)], buf)
        copies = [
            pltpu.async_copy(buf, out_hbm.at[i, pl.ds(base, rows)], sems[i])
            for i in range(b)
        ]
        for c in copies:
            c.wait()

    return k(w)


def kernel(x, embed_weight):
    b = x.shape[0]
    return _broadcast_table(embed_weight, b)
